# baseline (speedup 1.0000x reference)
import functools

import jax
import jax.numpy as jnp
from jax import lax
from jax.experimental import pallas as pl
from jax.experimental.pallas import tpu as pltpu
from jax.experimental.pallas import tpu_sc as plsc

NUM_CLUSTERS = 512
B = 1024
D = 128
L = 16
NS = 16
B_PER_W = B // NS

_SIGN = -2147483648


def _threefry_bits(k1, k2, x1):
    ks2 = k1 ^ k2 ^ jnp.int32(0x1BD11BDA)
    ks = (k1, k2, ks2)
    x0 = jnp.zeros_like(x1) + ks[0]
    x1 = x1 + ks[1]
    rots = ((13, 15, 26, 6), (17, 29, 16, 24))
    for i in range(5):
        for r in rots[i % 2]:
            x0 = x0 + x1
            x1 = (x1 << r) | lax.shift_right_logical(x1, 32 - r)
            x1 = x0 ^ x1
        x0 = x0 + ks[(i + 1) % 3]
        x1 = x1 + ks[(i + 2) % 3] + jnp.int32(i + 1)
    return x0 ^ x1


def _tc_sample(p_ref, out_ref):
    seed = p_ref[0, 0]
    k1 = lax.shift_right_logical(seed, jnp.int32(32))
    k2 = seed
    row = lax.broadcasted_iota(jnp.int32, (4, D), 0)
    col = lax.broadcasted_iota(jnp.int32, (4, D), 1)
    counts = row * D + col
    bits = _threefry_bits(jnp.full((4, D), k1, jnp.int32),
                          jnp.full((4, D), k2, jnp.int32), counts)
    packed = (bits & jnp.int32(-512)) | (jnp.int32(511) - counts)
    m = jnp.max(packed ^ jnp.int32(_SIGN))
    z = jnp.int32(511) - ((m ^ jnp.int32(_SIGN)) & jnp.int32(511))
    out_ref[...] = jnp.full((1, D), z, jnp.int32)


_tc_sample_call = pl.pallas_call(
    _tc_sample,
    out_shape=jax.ShapeDtypeStruct((1, D), jnp.int32),
    in_specs=[pl.BlockSpec(memory_space=pltpu.SMEM)],
)


def _sc_body(z_hbm, mus_hbm, sigmas_hbm, mu_out, sigma_out, z_v, sem):
    sid = lax.axis_index("s")
    base = sid * B_PER_W

    pltpu.sync_copy(z_hbm.at[0, pl.ds(0, L)], z_v)
    z = z_v[...][0]

    rows = pl.ds(base, B_PER_W)
    zsel = pl.ds(z, 1)
    one = pl.ds(0, 1)
    cp_mu = pltpu.async_copy(mus_hbm.at[rows, zsel, :],
                             mu_out.at[rows, one, :], sem)
    cp_sig = pltpu.async_copy(sigmas_hbm.at[rows, zsel, :],
                              sigma_out.at[rows, one, :], sem)
    cp_mu.wait()
    cp_sig.wait()


_sc_gather = functools.partial(
    pl.kernel,
    out_type=[
        jax.ShapeDtypeStruct((B, 1, D), jnp.float32),
        jax.ShapeDtypeStruct((B, 1, D), jnp.float32),
    ],
    mesh=plsc.VectorSubcoreMesh(core_axis_name="c", subcore_axis_name="s",
                                num_cores=1, num_subcores=NS),
    scratch_types=[
        pltpu.VMEM((L,), jnp.int32),
        pltpu.SemaphoreType.DMA,
    ],
)(_sc_body)


def kernel(p, mus, sigmas, pi):
    del pi
    p_arr = jnp.asarray(p, jnp.int32).reshape(1, 1)
    z_arr = _tc_sample_call(p_arr)
    mu_z, sigma_z = _sc_gather(z_arr, mus, sigmas)
    return (mu_z.reshape(B, D), sigma_z.reshape(B, D))


# speedup vs baseline: 2.4650x; 2.4650x over previous
import functools

import jax
import jax.numpy as jnp
from jax import lax
from jax.experimental import pallas as pl
from jax.experimental.pallas import tpu as pltpu
from jax.experimental.pallas import tpu_sc as plsc

NUM_CLUSTERS = 512
B = 1024
D = 128
L = 16
NS = 16
B_PER_W = B // NS

_SIGN = -2147483648


def _threefry_bits(k1, k2, x1):
    ks2 = k1 ^ k2 ^ jnp.int32(0x1BD11BDA)
    ks = (k1, k2, ks2)
    x0 = jnp.zeros_like(x1) + ks[0]
    x1 = x1 + ks[1]
    rots = ((13, 15, 26, 6), (17, 29, 16, 24))
    for i in range(5):
        for r in rots[i % 2]:
            x0 = x0 + x1
            x1 = (x1 << r) | lax.shift_right_logical(x1, 32 - r)
            x1 = x0 ^ x1
        x0 = x0 + ks[(i + 1) % 3]
        x1 = x1 + ks[(i + 2) % 3] + jnp.int32(i + 1)
    return x0 ^ x1


def _tc_sample(p_ref, out_ref):
    seed = p_ref[0, 0]
    k1 = lax.shift_right_logical(seed, jnp.int32(32))
    k2 = seed
    row = lax.broadcasted_iota(jnp.int32, (4, D), 0)
    col = lax.broadcasted_iota(jnp.int32, (4, D), 1)
    counts = row * D + col
    bits = _threefry_bits(jnp.full((4, D), k1, jnp.int32),
                          jnp.full((4, D), k2, jnp.int32), counts)
    packed = (bits & jnp.int32(-512)) | (jnp.int32(511) - counts)
    m = jnp.max(packed ^ jnp.int32(_SIGN))
    z = jnp.int32(511) - ((m ^ jnp.int32(_SIGN)) & jnp.int32(511))
    out_ref[...] = jnp.full((1, D), z, jnp.int32)


_tc_sample_call = pl.pallas_call(
    _tc_sample,
    out_shape=jax.ShapeDtypeStruct((1, D), jnp.int32),
    in_specs=[pl.BlockSpec(memory_space=pltpu.SMEM)],
)


def _sc_body(z_hbm, mus_hbm, sigmas_hbm, mu_out, sigma_out,
             z_v, mu_rows, sig_rows, sem_g, sem_s):
    sid = lax.axis_index("s")
    base = sid * B_PER_W

    pltpu.sync_copy(z_hbm.at[0, pl.ds(0, L)], z_v)
    z = z_v[...][0]

    rows = pl.ds(base, B_PER_W)
    zsel = pl.ds(z, 1)
    one = pl.ds(0, 1)
    g_mu = pltpu.async_copy(mus_hbm.at[rows, zsel, :], mu_rows, sem_g)
    g_sig = pltpu.async_copy(sigmas_hbm.at[rows, zsel, :], sig_rows, sem_g)
    g_mu.wait()
    s_mu = pltpu.async_copy(mu_rows, mu_out.at[rows, one, :], sem_s)
    g_sig.wait()
    s_sig = pltpu.async_copy(sig_rows, sigma_out.at[rows, one, :], sem_s)
    s_mu.wait()
    s_sig.wait()


_sc_gather = functools.partial(
    pl.kernel,
    out_type=[
        jax.ShapeDtypeStruct((B, 1, D), jnp.float32),
        jax.ShapeDtypeStruct((B, 1, D), jnp.float32),
    ],
    mesh=plsc.VectorSubcoreMesh(core_axis_name="c", subcore_axis_name="s",
                                num_cores=1, num_subcores=NS),
    scratch_types=[
        pltpu.VMEM((L,), jnp.int32),
        pltpu.VMEM((B_PER_W, 1, D), jnp.float32),
        pltpu.VMEM((B_PER_W, 1, D), jnp.float32),
        pltpu.SemaphoreType.DMA,
        pltpu.SemaphoreType.DMA,
    ],
)(_sc_body)


def kernel(p, mus, sigmas, pi):
    del pi
    p_arr = jnp.asarray(p, jnp.int32).reshape(1, 1)
    z_arr = _tc_sample_call(p_arr)
    mu_z, sigma_z = _sc_gather(z_arr, mus, sigmas)
    return (mu_z.reshape(B, D), sigma_z.reshape(B, D))
